# baseline (speedup 1.0000x reference)
import functools

import jax
import jax.numpy as jnp
from jax import lax
from jax.experimental import pallas as pl
from jax.experimental.pallas import tpu as pltpu
from jax.experimental.pallas import tpu_sc as plsc

_DEPTH = 22
_ATOMS = 5
_PLANES = _ATOMS * _DEPTH
_PLANE_ROWS = 2048
_ROWS = _PLANES * _PLANE_ROWS
_WROWS = 64
_GROUPS = _WROWS * 8


def _sc_body(x_hbm, o_hbm, strips, best, idx, oh):
    wid = lax.axis_index("s") * 2 + lax.axis_index("c")
    base = wid * _WROWS

    def init_body(g, c):
        r = g >> 3
        cc = (g & 7) * 16
        best[r, pl.ds(cc, 16)] = jnp.full((16,), -jnp.inf, jnp.float32)
        idx[r, pl.ds(cc, 16)] = jnp.zeros((16,), jnp.int32)
        return c

    lax.fori_loop(0, _GROUPS, init_body, 0)

    def p1_body(d, c):
        for a_ in range(_ATOMS):
            row0 = (a_ * _DEPTH + d) * _PLANE_ROWS + base
            pltpu.sync_copy(x_hbm.at[pl.ds(row0, _WROWS), :], strips.at[a_])

        def g_body(g, cc_):
            r = g >> 3
            cc = (g & 7) * 16
            s = (
                strips[0, r, pl.ds(cc, 16)]
                + strips[1, r, pl.ds(cc, 16)]
                + strips[2, r, pl.ds(cc, 16)]
                + strips[3, r, pl.ds(cc, 16)]
                + strips[4, r, pl.ds(cc, 16)]
            )
            b = best[r, pl.ds(cc, 16)]
            gt = s > b
            best[r, pl.ds(cc, 16)] = jnp.where(gt, s, b)
            iv = idx[r, pl.ds(cc, 16)]
            idx[r, pl.ds(cc, 16)] = jnp.where(gt, d, iv)
            return cc_

        lax.fori_loop(0, _GROUPS, g_body, 0)
        return c

    lax.fori_loop(0, _DEPTH, p1_body, 0)

    def p2_body(d, c):
        def g_body(g, cc_):
            r = g >> 3
            cc = (g & 7) * 16
            iv = idx[r, pl.ds(cc, 16)]
            oh[r, pl.ds(cc, 16)] = jnp.where(
                iv == d, jnp.float32(1.0), jnp.float32(0.0)
            )
            return cc_

        lax.fori_loop(0, _GROUPS, g_body, 0)
        for a_ in range(_ATOMS):
            row0 = (a_ * _DEPTH + d) * _PLANE_ROWS + base
            pltpu.sync_copy(oh, o_hbm.at[pl.ds(row0, _WROWS), :])
        return c

    lax.fori_loop(0, _DEPTH, p2_body, 0)


def kernel(inputs):
    b, l, a, d = inputs.shape
    x2 = (
        jnp.transpose(inputs, (2, 3, 0, 1))
        .reshape(_PLANES, b // 8, 8, l // 128, 128)
        .transpose(0, 1, 3, 2, 4)
        .reshape(_ROWS, 128)
    )
    mesh = plsc.VectorSubcoreMesh(core_axis_name="c", subcore_axis_name="s")
    f = pl.kernel(
        _sc_body,
        out_type=jax.ShapeDtypeStruct((_ROWS, 128), jnp.float32),
        mesh=mesh,
        scratch_types=[
            pltpu.VMEM((_ATOMS, _WROWS, 128), jnp.float32),
            pltpu.VMEM((_WROWS, 128), jnp.float32),
            pltpu.VMEM((_WROWS, 128), jnp.int32),
            pltpu.VMEM((_WROWS, 128), jnp.float32),
        ],
    )
    o2 = f(x2)
    return (
        o2.reshape(_PLANES, b // 8, l // 128, 8, 128)
        .transpose(0, 1, 3, 2, 4)
        .reshape(a, d, b, l)
        .transpose(2, 3, 0, 1)
    )


# speedup vs baseline: 1.6319x; 1.6319x over previous
import functools

import jax
import jax.numpy as jnp
from jax import lax
from jax.experimental import pallas as pl
from jax.experimental.pallas import tpu as pltpu
from jax.experimental.pallas import tpu_sc as plsc

_DEPTH = 22
_ATOMS = 5
_PLANES = _ATOMS * _DEPTH
_PLANE_ROWS = 2048
_ROWS = _PLANES * _PLANE_ROWS
_WROWS = 64
_GROUPS = _WROWS * 8


def _sc_body(x_hbm, o_hbm, strips, best, idx, oh, si0, si1, so0, so1):
    wid = lax.axis_index("s") * 2 + lax.axis_index("c")
    base = wid * _WROWS

    def in_copy(d, bank, sem):
        return [
            pltpu.make_async_copy(
                x_hbm.at[pl.ds((a_ * _DEPTH + d) * _PLANE_ROWS + base, _WROWS), :],
                strips.at[bank, a_],
                sem,
            )
            for a_ in range(_ATOMS)
        ]

    def out_copy(d, bank, sem):
        return [
            pltpu.make_async_copy(
                oh.at[bank],
                o_hbm.at[pl.ds((a_ * _DEPTH + d) * _PLANE_ROWS + base, _WROWS), :],
                sem,
            )
            for a_ in range(_ATOMS)
        ]

    def init_body(g, c):
        r = g >> 3
        cc = (g & 7) * 16
        best[r, pl.ds(cc, 16)] = jnp.full((16,), -jnp.inf, jnp.float32)
        idx[r, pl.ds(cc, 16)] = jnp.zeros((16,), jnp.int32)
        return c

    lax.fori_loop(0, _GROUPS, init_body, 0)

    def p1_compute(d, bank):
        def g_body(g, cc_):
            r = g >> 3
            cc = (g & 7) * 16
            s = (
                strips[bank, 0, r, pl.ds(cc, 16)]
                + strips[bank, 1, r, pl.ds(cc, 16)]
                + strips[bank, 2, r, pl.ds(cc, 16)]
                + strips[bank, 3, r, pl.ds(cc, 16)]
                + strips[bank, 4, r, pl.ds(cc, 16)]
            )
            b = best[r, pl.ds(cc, 16)]
            gt = s > b
            best[r, pl.ds(cc, 16)] = jnp.where(gt, s, b)
            iv = idx[r, pl.ds(cc, 16)]
            idx[r, pl.ds(cc, 16)] = jnp.where(gt, d, iv)
            return cc_

        lax.fori_loop(0, _GROUPS, g_body, 0)

    for c in in_copy(0, 0, si0):
        c.start()

    def p1_outer(i, c):
        d0 = 2 * i
        for cp in in_copy(d0 + 1, 1, si1):
            cp.start()
        for cp in in_copy(d0, 0, si0):
            cp.wait()
        p1_compute(d0, 0)

        @pl.when(d0 + 2 < _DEPTH)
        def _():
            for cp in in_copy(d0 + 2, 0, si0):
                cp.start()

        for cp in in_copy(d0 + 1, 1, si1):
            cp.wait()
        p1_compute(d0 + 1, 1)
        return c

    lax.fori_loop(0, _DEPTH // 2, p1_outer, 0)

    def p2_build(d, bank):
        def g_body(g, cc_):
            r = g >> 3
            cc = (g & 7) * 16
            iv = idx[r, pl.ds(cc, 16)]
            oh[bank, r, pl.ds(cc, 16)] = jnp.where(
                iv == d, jnp.float32(1.0), jnp.float32(0.0)
            )
            return cc_

        lax.fori_loop(0, _GROUPS, g_body, 0)

    def p2_outer(i, c):
        d0 = 2 * i

        @pl.when(i > 0)
        def _():
            for cp in out_copy(d0 - 2, 0, so0):
                cp.wait()

        p2_build(d0, 0)
        for cp in out_copy(d0, 0, so0):
            cp.start()

        @pl.when(i > 0)
        def _():
            for cp in out_copy(d0 - 1, 1, so1):
                cp.wait()

        p2_build(d0 + 1, 1)
        for cp in out_copy(d0 + 1, 1, so1):
            cp.start()
        return c

    lax.fori_loop(0, _DEPTH // 2, p2_outer, 0)
    for cp in out_copy(_DEPTH - 2, 0, so0):
        cp.wait()
    for cp in out_copy(_DEPTH - 1, 1, so1):
        cp.wait()


def kernel(inputs):
    b, l, a, d = inputs.shape
    x2 = (
        jnp.transpose(inputs, (2, 3, 0, 1))
        .reshape(_PLANES, b // 8, 8, l // 128, 128)
        .transpose(0, 1, 3, 2, 4)
        .reshape(_ROWS, 128)
    )
    mesh = plsc.VectorSubcoreMesh(core_axis_name="c", subcore_axis_name="s")
    f = pl.kernel(
        _sc_body,
        out_type=jax.ShapeDtypeStruct((_ROWS, 128), jnp.float32),
        mesh=mesh,
        scratch_types=[
            pltpu.VMEM((2, _ATOMS, _WROWS, 128), jnp.float32),
            pltpu.VMEM((_WROWS, 128), jnp.float32),
            pltpu.VMEM((_WROWS, 128), jnp.int32),
            pltpu.VMEM((2, _WROWS, 128), jnp.float32),
            pltpu.SemaphoreType.DMA,
            pltpu.SemaphoreType.DMA,
            pltpu.SemaphoreType.DMA,
            pltpu.SemaphoreType.DMA,
        ],
    )
    o2 = f(x2)
    return (
        o2.reshape(_PLANES, b // 8, l // 128, 8, 128)
        .transpose(0, 1, 3, 2, 4)
        .reshape(a, d, b, l)
        .transpose(2, 3, 0, 1)
    )


# speedup vs baseline: 3.2037x; 1.9632x over previous
import functools

import jax
import jax.numpy as jnp
from jax import lax
from jax.experimental import pallas as pl
from jax.experimental.pallas import tpu as pltpu
from jax.experimental.pallas import tpu_sc as plsc

_DEPTH = 22
_ATOMS = 5
_PLANES = _ATOMS * _DEPTH
_PLANE_ROWS = 2048
_ROWS = _PLANES * _PLANE_ROWS
_WROWS = 32


def _sc_body(x_hbm, o_hbm, strips, best, idx, oh, si0, si1, so0, so1):
    wid = lax.axis_index("s") * 2 + lax.axis_index("c")

    def in_pair(base, d0, bp, sem):
        cps = []
        for dd in range(2):
            for a_ in range(_ATOMS):
                cps.append(
                    pltpu.make_async_copy(
                        x_hbm.at[
                            pl.ds(
                                (a_ * _DEPTH + d0 + dd) * _PLANE_ROWS + base,
                                _WROWS,
                            ),
                            :,
                        ],
                        strips.at[2 * bp + dd, a_],
                        sem,
                    )
                )
        return cps

    def out_d(base, d, bank, sem):
        return [
            pltpu.make_async_copy(
                oh.at[bank],
                o_hbm.at[
                    pl.ds((a_ * _DEPTH + d) * _PLANE_ROWS + base, _WROWS), :
                ],
                sem,
            )
            for a_ in range(_ATOMS)
        ]

    def start(cps):
        for cp in cps:
            cp.start()

    def wait(cps):
        for cp in cps:
            cp.wait()

    def sum5(bk, r, cc):
        return (
            strips[bk, 0, r, pl.ds(cc, 16)]
            + strips[bk, 1, r, pl.ds(cc, 16)]
            + strips[bk, 2, r, pl.ds(cc, 16)]
            + strips[bk, 3, r, pl.ds(cc, 16)]
            + strips[bk, 4, r, pl.ds(cc, 16)]
        )

    def compute_pair(bp, d0, first):
        d0v = jnp.full((16,), d0, jnp.int32)

        def r_body(r, c):
            for k in range(8):
                cc = k * 16
                s0 = sum5(2 * bp, r, cc)
                s1 = sum5(2 * bp + 1, r, cc)
                gt1 = s1 > s0
                sm = jnp.where(gt1, s1, s0)
                dm = jnp.where(gt1, d0v + 1, d0v)
                if first:
                    best[r, pl.ds(cc, 16)] = sm
                    idx[r, pl.ds(cc, 16)] = dm
                else:
                    b = best[r, pl.ds(cc, 16)]
                    gt = sm > b
                    best[r, pl.ds(cc, 16)] = jnp.where(gt, sm, b)
                    iv = idx[r, pl.ds(cc, 16)]
                    idx[r, pl.ds(cc, 16)] = jnp.where(gt, dm, iv)
            return c

        lax.fori_loop(0, _WROWS, r_body, 0)

    def phase1(base):
        start(in_pair(base, 0, 0, si0))
        start(in_pair(base, 2, 1, si1))
        wait(in_pair(base, 0, 0, si0))
        compute_pair(0, 0, True)
        start(in_pair(base, 4, 0, si0))
        wait(in_pair(base, 2, 1, si1))
        compute_pair(1, jnp.int32(2), False)
        start(in_pair(base, 6, 1, si1))

        def body(i, c):
            d0 = 4 * i + 4
            wait(in_pair(base, d0, 0, si0))
            compute_pair(0, d0, False)
            start(in_pair(base, d0 + 4, 0, si0))
            wait(in_pair(base, d0 + 2, 1, si1))
            compute_pair(1, d0 + 2, False)

            @pl.when(d0 + 6 < _DEPTH)
            def _():
                start(in_pair(base, d0 + 6, 1, si1))

            return c

        lax.fori_loop(0, 4, body, 0)
        wait(in_pair(base, _DEPTH - 2, 0, si0))
        compute_pair(0, jnp.int32(_DEPTH - 2), False)

    def p2_build(d, bank):
        dv = jnp.full((16,), d, jnp.int32)

        def r_body(r, c):
            for k in range(8):
                cc = k * 16
                iv = idx[r, pl.ds(cc, 16)]
                oh[bank, r, pl.ds(cc, 16)] = jnp.where(
                    iv == dv, jnp.float32(1.0), jnp.float32(0.0)
                )
            return c

        lax.fori_loop(0, _WROWS, r_body, 0)

    def phase2(base):
        def body(i, c):
            d0 = 2 * i

            @pl.when(i > 0)
            def _():
                wait(out_d(base, d0 - 2, 0, so0))

            p2_build(d0, 0)
            start(out_d(base, d0, 0, so0))

            @pl.when(i > 0)
            def _():
                wait(out_d(base, d0 - 1, 1, so1))

            p2_build(d0 + 1, 1)
            start(out_d(base, d0 + 1, 1, so1))
            return c

        lax.fori_loop(0, _DEPTH // 2, body, 0)
        wait(out_d(base, _DEPTH - 2, 0, so0))
        wait(out_d(base, _DEPTH - 1, 1, so1))

    for h in range(2):
        base = wid * 64 + h * _WROWS
        phase1(base)
        phase2(base)


def kernel(inputs):
    b, l, a, d = inputs.shape
    x2 = (
        jnp.transpose(inputs, (2, 3, 0, 1))
        .reshape(_PLANES, b // 8, 8, l // 128, 128)
        .transpose(0, 1, 3, 2, 4)
        .reshape(_ROWS, 128)
    )
    mesh = plsc.VectorSubcoreMesh(core_axis_name="c", subcore_axis_name="s")
    f = pl.kernel(
        _sc_body,
        out_type=jax.ShapeDtypeStruct((_ROWS, 128), jnp.float32),
        mesh=mesh,
        scratch_types=[
            pltpu.VMEM((4, _ATOMS, _WROWS, 128), jnp.float32),
            pltpu.VMEM((_WROWS, 128), jnp.float32),
            pltpu.VMEM((_WROWS, 128), jnp.int32),
            pltpu.VMEM((2, _WROWS, 128), jnp.float32),
            pltpu.SemaphoreType.DMA,
            pltpu.SemaphoreType.DMA,
            pltpu.SemaphoreType.DMA,
            pltpu.SemaphoreType.DMA,
        ],
    )
    o2 = f(x2)
    return (
        o2.reshape(_PLANES, b // 8, l // 128, 8, 128)
        .transpose(0, 1, 3, 2, 4)
        .reshape(a, d, b, l)
        .transpose(2, 3, 0, 1)
    )
